# Initial kernel scaffold; baseline (speedup 1.0000x reference)
import jax
import jax.numpy as jnp
from jax.experimental import pallas as pl


def kernel(x, codebooks):
    raise NotImplementedError("write your pallas kernel here")


# speedup vs baseline: 1.6773x; 1.6773x over previous
import jax
import jax.numpy as jnp
from jax.experimental import pallas as pl
from jax.experimental.pallas import tpu as pltpu

_NQ = 8
_K = 1024
_D = 256
_TILE = 512


def _rvq_tile_kernel(x_ref, cb_ref, out_ref, ind_ref):
    r = x_ref[0]
    qsum = jnp.zeros_like(r)
    for i in range(_NQ):
        e = cb_ref[i]
        c2 = jnp.sum(e * e, axis=1, keepdims=True)
        s = c2 - 2.0 * jax.lax.dot_general(
            e, r, (((1,), (0,)), ((), ())),
            preferred_element_type=jnp.float32)
        ind = jnp.argmin(s, axis=0)
        oh = (jax.lax.broadcasted_iota(jnp.int32, (_K, r.shape[1]), 0)
              == ind[None, :]).astype(jnp.float32)
        q = jax.lax.dot_general(
            e, oh, (((0,), (0,)), ((), ())),
            preferred_element_type=jnp.float32,
            precision=jax.lax.Precision.HIGHEST)
        r = r - q
        qsum = qsum + q
        ind_ref[0, 0, i, :] = ind
    out_ref[0] = qsum


def kernel(x, codebooks):
    b, d, n = x.shape
    nt = n // _TILE
    out, ind = pl.pallas_call(
        _rvq_tile_kernel,
        grid=(b, nt),
        in_specs=[
            pl.BlockSpec((1, d, _TILE), lambda ib, it: (ib, 0, it)),
            pl.BlockSpec((_NQ, _K, _D), lambda ib, it: (0, 0, 0)),
        ],
        out_specs=[
            pl.BlockSpec((1, d, _TILE), lambda ib, it: (ib, 0, it)),
            pl.BlockSpec((1, 1, _NQ, _TILE), lambda ib, it: (ib, it, 0, 0)),
        ],
        out_shape=[
            jax.ShapeDtypeStruct((b, d, n), jnp.float32),
            jax.ShapeDtypeStruct((b, nt, _NQ, _TILE), jnp.int32),
        ],
        compiler_params=pltpu.CompilerParams(
            dimension_semantics=("parallel", "parallel")),
    )(x, codebooks)
    out_indices = ind.transpose(2, 0, 1, 3).reshape(_NQ, b, n)
    return out, out_indices


# speedup vs baseline: 2.6129x; 1.5578x over previous
import jax
import jax.numpy as jnp
from jax.experimental import pallas as pl
from jax.experimental.pallas import tpu as pltpu

_NQ = 8
_K = 1024
_D = 256
_TILE = 512


def _split_kernel(cb_ref, hi_ref, lo_ref, lo2_ref):
    c = cb_ref[...]
    hi = c.astype(jnp.bfloat16)
    rem = c - hi.astype(jnp.float32)
    lo = rem.astype(jnp.bfloat16)
    lo2 = (rem - lo.astype(jnp.float32)).astype(jnp.bfloat16)
    hi_ref[...] = hi
    lo_ref[...] = lo
    lo2_ref[...] = lo2


def _rvq_tile_kernel(x_ref, cb_ref, hi_ref, lo_ref, lo2_ref, out_ref, ind_ref):
    r = x_ref[0]
    qsum = jnp.zeros_like(r)
    for i in range(_NQ):
        e = cb_ref[i]
        c2 = jnp.sum(e * e, axis=1, keepdims=True)
        s = c2 - 2.0 * jax.lax.dot_general(
            e, r, (((1,), (0,)), ((), ())),
            preferred_element_type=jnp.float32)
        ind = jnp.argmin(s, axis=0)
        oh = (jax.lax.broadcasted_iota(jnp.int32, (_K, r.shape[1]), 0)
              == ind[None, :]).astype(jnp.bfloat16)
        q = jnp.zeros((_D, r.shape[1]), jnp.float32)
        for part in (hi_ref, lo_ref, lo2_ref):
            q = q + jax.lax.dot_general(
                part[i], oh, (((0,), (0,)), ((), ())),
                preferred_element_type=jnp.float32)
        r = r - q
        qsum = qsum + q
        ind_ref[0, 0, i, :] = ind
    out_ref[0] = qsum


def kernel(x, codebooks):
    b, d, n = x.shape
    nt = n // _TILE
    cb_hi, cb_lo, cb_lo2 = pl.pallas_call(
        _split_kernel,
        out_shape=[jax.ShapeDtypeStruct((_NQ, _K, _D), jnp.bfloat16)] * 3,
    )(codebooks)
    out, ind = pl.pallas_call(
        _rvq_tile_kernel,
        grid=(b, nt),
        in_specs=[
            pl.BlockSpec((1, d, _TILE), lambda ib, it: (ib, 0, it)),
            pl.BlockSpec((_NQ, _K, _D), lambda ib, it: (0, 0, 0)),
            pl.BlockSpec((_NQ, _K, _D), lambda ib, it: (0, 0, 0)),
            pl.BlockSpec((_NQ, _K, _D), lambda ib, it: (0, 0, 0)),
            pl.BlockSpec((_NQ, _K, _D), lambda ib, it: (0, 0, 0)),
        ],
        out_specs=[
            pl.BlockSpec((1, d, _TILE), lambda ib, it: (ib, 0, it)),
            pl.BlockSpec((1, 1, _NQ, _TILE), lambda ib, it: (ib, it, 0, 0)),
        ],
        out_shape=[
            jax.ShapeDtypeStruct((b, d, n), jnp.float32),
            jax.ShapeDtypeStruct((b, nt, _NQ, _TILE), jnp.int32),
        ],
        compiler_params=pltpu.CompilerParams(
            dimension_semantics=("parallel", "parallel")),
    )(x, codebooks, cb_hi, cb_lo, cb_lo2)
    out_indices = ind.transpose(2, 0, 1, 3).reshape(_NQ, b, n)
    return out, out_indices


# speedup vs baseline: 3.3487x; 1.2816x over previous
import jax
import jax.numpy as jnp
from jax.experimental import pallas as pl
from jax.experimental.pallas import tpu as pltpu

_NQ = 8
_K = 1024
_D = 256
_TILE = 1024


def _prep_kernel(cb_ref, em2_ref, hi_ref, lo_ref, lo2_ref, c2_ref):
    c = cb_ref[...]
    em2_ref[...] = -2.0 * c
    hi = c.astype(jnp.bfloat16)
    rem = c - hi.astype(jnp.float32)
    lo = rem.astype(jnp.bfloat16)
    lo2 = (rem - lo.astype(jnp.float32)).astype(jnp.bfloat16)
    hi_ref[...] = hi
    lo_ref[...] = lo
    lo2_ref[...] = lo2
    c2_ref[...] = jnp.sum(c * c, axis=-1, keepdims=True)


def _rvq_tile_kernel(x_ref, em2_ref, hi_ref, lo_ref, lo2_ref, c2_ref, out_ref,
                     ind_ref):
    r = x_ref[0]
    t = r.shape[1]
    qsum = jnp.zeros_like(r)
    for i in range(_NQ):
        s = c2_ref[i] + jax.lax.dot_general(
            em2_ref[i], r, (((1,), (0,)), ((), ())),
            preferred_element_type=jnp.float32)
        ind = jnp.argmin(s, axis=0)
        oh = (jax.lax.broadcasted_iota(jnp.int32, (_K, t), 0)
              == ind[None, :]).astype(jnp.bfloat16)
        q = jnp.zeros((_D, t), jnp.float32)
        for part in (hi_ref, lo_ref, lo2_ref):
            q = q + jax.lax.dot_general(
                part[i], oh, (((0,), (0,)), ((), ())),
                preferred_element_type=jnp.float32)
        r = r - q
        qsum = qsum + q
        ind_ref[0, 0, i, :] = ind
    out_ref[0] = qsum


def kernel(x, codebooks):
    b, d, n = x.shape
    nt = n // _TILE
    em2, cb_hi, cb_lo, cb_lo2, c2 = pl.pallas_call(
        _prep_kernel,
        out_shape=[
            jax.ShapeDtypeStruct((_NQ, _K, _D), jnp.float32),
            jax.ShapeDtypeStruct((_NQ, _K, _D), jnp.bfloat16),
            jax.ShapeDtypeStruct((_NQ, _K, _D), jnp.bfloat16),
            jax.ShapeDtypeStruct((_NQ, _K, _D), jnp.bfloat16),
            jax.ShapeDtypeStruct((_NQ, _K, 1), jnp.float32),
        ],
    )(codebooks)
    out, ind = pl.pallas_call(
        _rvq_tile_kernel,
        grid=(b, nt),
        in_specs=[
            pl.BlockSpec((1, d, _TILE), lambda ib, it: (ib, 0, it)),
            pl.BlockSpec((_NQ, _K, _D), lambda ib, it: (0, 0, 0)),
            pl.BlockSpec((_NQ, _K, _D), lambda ib, it: (0, 0, 0)),
            pl.BlockSpec((_NQ, _K, _D), lambda ib, it: (0, 0, 0)),
            pl.BlockSpec((_NQ, _K, _D), lambda ib, it: (0, 0, 0)),
            pl.BlockSpec((_NQ, _K, 1), lambda ib, it: (0, 0, 0)),
        ],
        out_specs=[
            pl.BlockSpec((1, d, _TILE), lambda ib, it: (ib, 0, it)),
            pl.BlockSpec((1, 1, _NQ, _TILE), lambda ib, it: (ib, it, 0, 0)),
        ],
        out_shape=[
            jax.ShapeDtypeStruct((b, d, n), jnp.float32),
            jax.ShapeDtypeStruct((b, nt, _NQ, _TILE), jnp.int32),
        ],
        compiler_params=pltpu.CompilerParams(
            dimension_semantics=("parallel", "parallel")),
    )(x, em2, cb_hi, cb_lo, cb_lo2, c2)
    out_indices = ind.transpose(2, 0, 1, 3).reshape(_NQ, b, n)
    return out, out_indices


# speedup vs baseline: 3.4247x; 1.0227x over previous
import jax
import jax.numpy as jnp
from jax.experimental import pallas as pl
from jax.experimental.pallas import tpu as pltpu

_NQ = 8
_K = 1024
_D = 256
_TILE = 2048


def _prep_kernel(cb_ref, em2_ref, hi_ref, lo_ref, lo2_ref, c2_ref):
    c = cb_ref[...]
    em2_ref[...] = -2.0 * c
    hi = c.astype(jnp.bfloat16)
    rem = c - hi.astype(jnp.float32)
    lo = rem.astype(jnp.bfloat16)
    lo2 = (rem - lo.astype(jnp.float32)).astype(jnp.bfloat16)
    hi_ref[...] = hi
    lo_ref[...] = lo
    lo2_ref[...] = lo2
    c2_ref[...] = jnp.sum(c * c, axis=-1, keepdims=True)


def _rvq_tile_kernel(x_ref, em2_ref, hi_ref, lo_ref, lo2_ref, c2_ref, out_ref,
                     ind_ref):
    r = x_ref[0]
    t = r.shape[1]
    qsum = jnp.zeros_like(r)
    for i in range(_NQ):
        s = c2_ref[i] + jax.lax.dot_general(
            em2_ref[i], r, (((1,), (0,)), ((), ())),
            preferred_element_type=jnp.float32)
        ind = jnp.argmin(s, axis=0)
        oh = (jax.lax.broadcasted_iota(jnp.int32, (_K, t), 0)
              == ind[None, :]).astype(jnp.bfloat16)
        q = jnp.zeros((_D, t), jnp.float32)
        for part in (hi_ref, lo_ref, lo2_ref):
            q = q + jax.lax.dot_general(
                part[i], oh, (((0,), (0,)), ((), ())),
                preferred_element_type=jnp.float32)
        r = r - q
        qsum = qsum + q
        ind_ref[0, 0, i, :] = ind
    out_ref[0] = qsum


def kernel(x, codebooks):
    b, d, n = x.shape
    nt = n // _TILE
    em2, cb_hi, cb_lo, cb_lo2, c2 = pl.pallas_call(
        _prep_kernel,
        out_shape=[
            jax.ShapeDtypeStruct((_NQ, _K, _D), jnp.float32),
            jax.ShapeDtypeStruct((_NQ, _K, _D), jnp.bfloat16),
            jax.ShapeDtypeStruct((_NQ, _K, _D), jnp.bfloat16),
            jax.ShapeDtypeStruct((_NQ, _K, _D), jnp.bfloat16),
            jax.ShapeDtypeStruct((_NQ, _K, 1), jnp.float32),
        ],
    )(codebooks)
    out, ind = pl.pallas_call(
        _rvq_tile_kernel,
        grid=(b, nt),
        in_specs=[
            pl.BlockSpec((1, d, _TILE), lambda ib, it: (ib, 0, it)),
            pl.BlockSpec((_NQ, _K, _D), lambda ib, it: (0, 0, 0)),
            pl.BlockSpec((_NQ, _K, _D), lambda ib, it: (0, 0, 0)),
            pl.BlockSpec((_NQ, _K, _D), lambda ib, it: (0, 0, 0)),
            pl.BlockSpec((_NQ, _K, _D), lambda ib, it: (0, 0, 0)),
            pl.BlockSpec((_NQ, _K, 1), lambda ib, it: (0, 0, 0)),
        ],
        out_specs=[
            pl.BlockSpec((1, d, _TILE), lambda ib, it: (ib, 0, it)),
            pl.BlockSpec((1, 1, _NQ, _TILE), lambda ib, it: (ib, it, 0, 0)),
        ],
        out_shape=[
            jax.ShapeDtypeStruct((b, d, n), jnp.float32),
            jax.ShapeDtypeStruct((b, nt, _NQ, _TILE), jnp.int32),
        ],
        compiler_params=pltpu.CompilerParams(
            dimension_semantics=("parallel", "parallel")),
    )(x, em2, cb_hi, cb_lo, cb_lo2, c2)
    out_indices = ind.transpose(2, 0, 1, 3).reshape(_NQ, b, n)
    return out, out_indices
